# baseline (speedup 1.0000x reference)
import functools

import jax
import jax.numpy as jnp
from jax import lax
from jax.experimental import pallas as pl
from jax.experimental.pallas import tpu as pltpu
from jax.experimental.pallas import tpu_sc as plsc

EMB = 32
LANES = 16
IDXW = 128
QW = 512
KD = QW // IDXW
B = 4096
NQ = B // QW


@functools.lru_cache(maxsize=None)
def _make_gather(n_planes: int, vocab: int):
    info = plsc.get_sparse_core_info()
    nw = info.num_cores * info.num_subcores
    n_c = (n_planes + nw - 1) // nw
    rows_per_plane = B // IDXW

    mesh = plsc.VectorSubcoreMesh(core_axis_name="c", subcore_axis_name="s")

    @functools.partial(
        pl.kernel,
        mesh=mesh,
        out_type=jax.ShapeDtypeStruct((n_planes, EMB, B), jnp.float32),
        scratch_types=[
            pltpu.VMEM((rows_per_plane, IDXW), jnp.int32),
            pltpu.VMEM((2, QW, EMB), jnp.float32),
            pltpu.VMEM((2, EMB, QW), jnp.float32),
            pltpu.SemaphoreType.DMA((2,)),
            pltpu.SemaphoreType.DMA((2,)),
        ],
        compiler_params=pltpu.CompilerParams(
            use_tc_tiling_on_sc=False, needs_layout_passes=False),
    )
    def gather(idx_hbm, tbl_hbm, out_hbm, idx_v, gbuf, obuf, gsem, osem):
        wid = lax.axis_index("s") * info.num_cores + lax.axis_index("c")
        iota = lax.iota(jnp.int32, LANES)

        def gcp(s, j, r):
            return pltpu.make_async_copy(
                tbl_hbm.at[idx_v.at[r]],
                gbuf.at[s, pl.ds(j * IDXW, IDXW)],
                gsem.at[s])

        def ocp(t, q, s):
            return pltpu.make_async_copy(
                obuf.at[s],
                out_hbm.at[t, :, pl.ds(q * QW, QW)],
                osem.at[s])

        def plane(c, carry):
            t = c * nw + wid

            @pl.when(t < n_planes)
            def _():
                pltpu.sync_copy(
                    idx_hbm.at[pl.ds(t * rows_per_plane, rows_per_plane)],
                    idx_v)
                for j in range(KD):
                    gcp(0, j, j).start()
                for q in range(NQ):
                    s = q % 2
                    if q + 1 < NQ:
                        for j in range(KD):
                            gcp(1 - s, j, (q + 1) * KD + j).start()
                    for j in range(KD):
                        gcp(s, j, q * KD + j).wait()
                    if q >= 2:
                        ocp(t, q, s).wait()
                    else:
                        @pl.when(c > 0)
                        def _():
                            ocp(t, q, s).wait()

                    def tbody(wg, tcarry):
                        w_idx = iota + wg * LANES
                        for e in range(EMB):
                            e_idx = jnp.full((LANES,), e, jnp.int32)
                            val = plsc.load_gather(gbuf.at[s], [w_idx, e_idx])
                            obuf[s, e, pl.ds(wg * LANES, LANES)] = val
                        return tcarry

                    lax.fori_loop(0, QW // LANES, tbody, 0)
                    ocp(t, q, s).start()

            return carry

        lax.fori_loop(0, n_c, plane, 0)
        ocp(0, 0, 0).wait()
        ocp(0, 0, 1).wait()

    return gather


def kernel(x, table):
    b, h = x.shape
    idx = jnp.asarray(x, jnp.int32).T.reshape(h * b // IDXW, IDXW)
    out = _make_gather(h, table.shape[0])(idx, table)
    return jnp.transpose(out, (2, 0, 1))


# speedup vs baseline: 1.7330x; 1.7330x over previous
import functools

import jax
import jax.numpy as jnp
from jax import lax
from jax.experimental import pallas as pl
from jax.experimental.pallas import tpu as pltpu
from jax.experimental.pallas import tpu_sc as plsc

EMB = 32
LANES = 16
IDXW = 128
QW = 512
KD = QW // IDXW
B = 4096
NQ = B // QW


@functools.lru_cache(maxsize=None)
def _make_gather(n_planes: int, vocab: int):
    info = plsc.get_sparse_core_info()
    nw = info.num_cores * info.num_subcores
    n_c = (n_planes + nw - 1) // nw
    rows_per_plane = B // IDXW

    mesh = plsc.VectorSubcoreMesh(core_axis_name="c", subcore_axis_name="s")

    @functools.partial(
        pl.kernel,
        mesh=mesh,
        out_type=jax.ShapeDtypeStruct((n_planes, EMB, B), jnp.float32),
        scratch_types=[
            pltpu.VMEM((rows_per_plane, IDXW), jnp.int32),
            pltpu.VMEM((2, QW, EMB), jnp.float32),
            pltpu.VMEM((2, EMB, QW + 1), jnp.float32),
            pltpu.SemaphoreType.DMA((2,)),
            pltpu.SemaphoreType.DMA((2,)),
        ],
        compiler_params=pltpu.CompilerParams(
            use_tc_tiling_on_sc=False, needs_layout_passes=False),
    )
    def gather(idx_hbm, tbl_hbm, out_hbm, idx_v, gbuf, obuf, gsem, osem):
        wid = lax.axis_index("s") * info.num_cores + lax.axis_index("c")
        iota = lax.iota(jnp.int32, LANES)

        def gcp(s, j, r):
            return pltpu.make_async_copy(
                tbl_hbm.at[idx_v.at[r]],
                gbuf.at[s, pl.ds(j * IDXW, IDXW)],
                gsem.at[s])

        def ocp(t, q, s):
            return pltpu.make_async_copy(
                obuf.at[s, :, pl.ds(0, QW)],
                out_hbm.at[t, :, pl.ds(q * QW, QW)],
                osem.at[s])

        def plane(c, carry):
            t = c * nw + wid

            @pl.when(t < n_planes)
            def _():
                pltpu.sync_copy(
                    idx_hbm.at[pl.ds(t * rows_per_plane, rows_per_plane)],
                    idx_v)
                for j in range(KD):
                    gcp(0, j, j).start()
                for q in range(NQ):
                    s = q % 2
                    if q + 1 < NQ:
                        for j in range(KD):
                            gcp(1 - s, j, (q + 1) * KD + j).start()
                    for j in range(KD):
                        gcp(s, j, q * KD + j).wait()
                    if q >= 2:
                        ocp(t, q, s).wait()
                    else:
                        @pl.when(c > 0)
                        def _():
                            ocp(t, q, s).wait()

                    e_lo = iota
                    e_hi = iota + LANES

                    def tbody(wb, tcarry):
                        for u in range(8):
                            w = wb * 8 + u
                            w_vec = jnp.full((LANES,), 0, jnp.int32) + w
                            lo = gbuf[s, w, pl.ds(0, LANES)]
                            hi = gbuf[s, w, pl.ds(LANES, LANES)]
                            plsc.store_scatter(obuf.at[s], [e_lo, w_vec], lo)
                            plsc.store_scatter(obuf.at[s], [e_hi, w_vec], hi)
                        return tcarry

                    lax.fori_loop(0, QW // 8, tbody, 0)
                    ocp(t, q, s).start()

            return carry

        lax.fori_loop(0, n_c, plane, 0)
        ocp(0, 0, 0).wait()
        ocp(0, 0, 1).wait()

    return gather


def kernel(x, table):
    b, h = x.shape
    idx = jnp.asarray(x, jnp.int32).T.reshape(h * b // IDXW, IDXW)
    out = _make_gather(h, table.shape[0])(idx, table)
    return jnp.transpose(out, (2, 0, 1))


# speedup vs baseline: 1.8739x; 1.0814x over previous
import functools

import jax
import jax.numpy as jnp
from jax import lax
from jax.experimental import pallas as pl
from jax.experimental.pallas import tpu as pltpu
from jax.experimental.pallas import tpu_sc as plsc

EMB = 32
LANES = 16
IDXW = 128
QW = 512
KD = QW // IDXW
B = 4096
NQ = B // QW


@functools.lru_cache(maxsize=None)
def _make_gather(n_planes: int, vocab: int):
    info = plsc.get_sparse_core_info()
    nw = info.num_cores * info.num_subcores
    n_c = (n_planes + nw - 1) // nw
    rows_per_plane = B // IDXW

    mesh = plsc.VectorSubcoreMesh(core_axis_name="c", subcore_axis_name="s")

    @functools.partial(
        pl.kernel,
        mesh=mesh,
        out_type=jax.ShapeDtypeStruct((n_planes, EMB, B), jnp.float32),
        scratch_types=[
            pltpu.VMEM((rows_per_plane, IDXW), jnp.int32),
            pltpu.VMEM((2, QW, EMB), jnp.float32),
            pltpu.VMEM((2, EMB, QW + 1), jnp.float32),
            pltpu.SemaphoreType.DMA((2,)),
            pltpu.SemaphoreType.DMA((2,)),
        ],
        compiler_params=pltpu.CompilerParams(
            use_tc_tiling_on_sc=False, needs_layout_passes=False),
    )
    def gather(idx_hbm, tbl_hbm, out_hbm, idx_v, gbuf, obuf, gsem, osem):
        wid = lax.axis_index("s") * info.num_cores + lax.axis_index("c")
        iota = lax.iota(jnp.int32, LANES)

        def gcp(s, j, r):
            return pltpu.make_async_copy(
                tbl_hbm.at[idx_v.at[r]],
                gbuf.at[s, pl.ds(j * IDXW, IDXW)],
                gsem.at[s])

        def ocp(t, q, s):
            return pltpu.make_async_copy(
                obuf.at[s, :, pl.ds(0, QW)],
                out_hbm.at[t, :, pl.ds(q * QW, QW)],
                osem.at[s])

        def plane(c, carry):
            t = c * nw + wid

            @pl.when(t < n_planes)
            def _():
                pltpu.sync_copy(
                    idx_hbm.at[pl.ds(t * rows_per_plane, rows_per_plane)],
                    idx_v)
                for j in range(KD):
                    gcp(0, j, j).start()
                for q in range(NQ):
                    s = q % 2
                    if q + 1 < NQ:
                        for j in range(KD):
                            gcp(1 - s, j, (q + 1) * KD + j).start()
                    for j in range(KD):
                        gcp(s, j, q * KD + j).wait()
                    if q >= 2:
                        ocp(t, q, s).wait()
                    else:
                        @pl.when(c > 0)
                        def _():
                            ocp(t, q, s).wait()

                    e_lo = iota
                    e_hi = iota + LANES

                    @plsc.parallel_loop(0, QW, unroll=8)
                    def _(w):
                        w_vec = jnp.full((LANES,), 0, jnp.int32) + w
                        lo = gbuf[s, w, pl.ds(0, LANES)]
                        hi = gbuf[s, w, pl.ds(LANES, LANES)]
                        plsc.store_scatter(obuf.at[s], [e_lo, w_vec], lo)
                        plsc.store_scatter(obuf.at[s], [e_hi, w_vec], hi)
                    ocp(t, q, s).start()

            return carry

        lax.fori_loop(0, n_c, plane, 0)
        ocp(0, 0, 0).wait()
        ocp(0, 0, 1).wait()

    return gather


def kernel(x, table):
    b, h = x.shape
    idx = jnp.asarray(x, jnp.int32).T.reshape(h * b // IDXW, IDXW)
    out = _make_gather(h, table.shape[0])(idx, table)
    return jnp.transpose(out, (2, 0, 1))
